# baseline (speedup 1.0000x reference)
import jax
import jax.numpy as jnp
import numpy as np
from jax.experimental import pallas as pl
from jax.experimental.pallas import tpu as pltpu

DIM = 512
NUM_SLOTS = 64
CHUNK = 64
EPS = 1e-5
TILE = 256
TOK = 2048


def _softmax0(logits):
    m = jnp.max(logits, axis=0, keepdims=True)
    e = jnp.exp(logits - m)
    return e / jnp.sum(e, axis=0, keepdims=True)


def _fused_kernel(x_ref, mkq_ref, bkq_ref, wv_ref, bv_ref,
                  mask_ref, wo_ref, bo_ref, o_ref):
    bf = jnp.bfloat16
    ns = NUM_SLOTS
    x = x_ref[...]
    xb = x.astype(bf)
    dn_tab = (((0,), (1,)), ((), ()))
    l2 = jax.lax.dot_general(mkq_ref[...], xb, dn_tab,
                             preferred_element_type=jnp.float32) + bkq_ref[...]
    wwt = _softmax0(l2[:ns]).astype(bf)
    rwt = _softmax0(l2[ns:]).astype(bf)
    vb = jnp.dot(xb, wv_ref[...],
                 preferred_element_type=jnp.float32).astype(bf) + bv_ref[...]

    dn_ta = (((0,), (0,)), ((), ()))
    lncs = []
    for h in range(TOK // TILE):
        lo, hi = h * TILE, (h + 1) * TILE
        a = jax.lax.dot_general(rwt[:, lo:hi], wwt[:, lo:hi], dn_ta,
                                preferred_element_type=jnp.float32)
        ret = jnp.dot(a.astype(bf) * mask_ref[...], vb[lo:hi, :],
                      preferred_element_type=jnp.float32)
        mu = jnp.mean(ret, axis=-1, keepdims=True)
        msq = jnp.mean(ret * ret, axis=-1, keepdims=True)
        rs = jax.lax.rsqrt(msq - mu * mu + EPS)
        lncs.append((ret.astype(bf) - mu.astype(bf)) * rs.astype(bf))
    lnc = jnp.concatenate(lncs, axis=0)
    out = jnp.dot(lnc, wo_ref[...],
                  preferred_element_type=jnp.float32) + bo_ref[...]
    o_ref[...] = x + out


def kernel(x, slot_keys, Wk, bk, Wq, bq, Wv, bv, scale, ln_g, ln_b, Wo, bo):
    b, s, d = x.shape
    n = b * s
    ns = slot_keys.shape[0]
    x2 = x.reshape(n, d)
    bf = jnp.bfloat16
    hi = jax.lax.Precision.HIGHEST

    sc = scale[0]
    stk = jnp.concatenate([Wk, Wq, bk[None, :], bq[None, :]], axis=0)
    f = sc * jnp.dot(stk, slot_keys.T, precision=hi)
    mkq = jnp.concatenate([f[:d], f[d:2 * d]], axis=1)
    bkq = jnp.concatenate([f[2 * d], f[2 * d + 1]], axis=0)
    wo_eff = ln_g[:, None] * Wo
    bo_eff = jnp.dot(ln_b, Wo, precision=hi) + bo

    r = np.arange(TILE)
    mask = ((r[:, None] // CHUNK == r[None, :] // CHUNK)
            & (r[None, :] <= r[:, None]))
    mask = jnp.asarray(mask).astype(jnp.bfloat16)

    full = lambda i: (0, 0)
    wspec = pl.BlockSpec((d, d), full)
    out = pl.pallas_call(
        _fused_kernel,
        out_shape=jax.ShapeDtypeStruct((n, d), x.dtype),
        grid=(n // TOK,),
        in_specs=[
            pl.BlockSpec((TOK, d), lambda i: (i, 0)),
            pl.BlockSpec((d, 2 * ns), full),
            pl.BlockSpec((2 * ns, 1), full),
            wspec, pl.BlockSpec((1, d), full),
            pl.BlockSpec((TILE, TILE), full),
            wspec, pl.BlockSpec((1, d), full),
        ],
        out_specs=pl.BlockSpec((TOK, d), lambda i: (i, 0)),
        compiler_params=pltpu.CompilerParams(
            dimension_semantics=("parallel",),
            vmem_limit_bytes=52 * 1024 * 1024,
        ),
        name="slot_memory_phasor",
    )(x2, mkq.astype(bf), bkq.reshape(2 * ns, 1), Wv.astype(bf),
      bv.astype(bf).reshape(1, d), mask, wo_eff.astype(bf),
      bo_eff.reshape(1, d))
    return out.reshape(b, s, d)
